# baseline (speedup 1.0000x reference)
import jax
import jax.numpy as jnp
from jax import lax
from jax.experimental import pallas as pl
from jax.experimental.pallas import tpu as pltpu
from jax.experimental.pallas import tpu_sc as plsc

_BATCH = 64
_NUM_POS = 196
_HIDDEN = 768
_P_OUT = _NUM_POS + 1

_LANES = 16
_NB = 2
_NGROUPS = _BATCH // _NB
_NBUF = 3
_VECS = _HIDDEN // _LANES


def _body(feats_hbm, qw_hbm, pw_hbm, out_hbm,
          buf0, buf1, buf2, pw_buf, small, tail_out, tail_pw, qw_buf,
          si0, si1, si2, so0, so1, so2):
    bufs = [buf0, buf1, buf2]
    sems_in = [si0, si1, si2]
    sems_out = [so0, so1, so2]
    c = lax.axis_index("c")
    s = lax.axis_index("s")
    wid = s * 2 + c

    @pl.when(jnp.logical_and(wid >= 1, wid < 24))
    def _interior():
        r0 = wid * 8
        pltpu.sync_copy(pw_hbm.at[pl.ds(r0, 8), :], pw_buf)

        def in_dma(g):
            return pltpu.async_copy(
                feats_hbm.at[pl.ds(g * _NB, _NB), pl.ds(r0 - 8, 16), :],
                bufs[g % _NBUF], sems_in[g % _NBUF])

        def out_dma(g):
            return pltpu.async_copy(
                bufs[g % _NBUF].at[:, pl.ds(0, 8), :],
                out_hbm.at[pl.ds(g * _NB, _NB), pl.ds(r0, 8), :],
                sems_out[g % _NBUF])

        def compute(g):
            buf = bufs[g % _NBUF]

            def vec_step(v, _):
                co = v * _LANES
                for r in range(8):
                    pwv = pw_buf[r, pl.ds(co, _LANES)]
                    for b in range(_NB):
                        buf[b, r, pl.ds(co, _LANES)] = (
                            buf[b, r + 7, pl.ds(co, _LANES)] + pwv)
                return 0

            lax.fori_loop(0, _VECS, vec_step, 0)

        h_in = [None] * _NGROUPS
        h_out = [None] * _NGROUPS
        h_in[0] = in_dma(0)
        for g in range(_NGROUPS):
            if g >= 2:
                h_out[g - 2].wait()
            if g + 1 < _NGROUPS:
                h_in[g + 1] = in_dma(g + 1)
            h_in[g].wait()
            compute(g)
            h_out[g] = out_dma(g)
        h_out[_NGROUPS - 2].wait()
        h_out[_NGROUPS - 1].wait()

    @pl.when(wid == 0)
    def _head():
        pltpu.sync_copy(pw_hbm.at[pl.ds(0, 8), :], pw_buf)
        pltpu.sync_copy(qw_hbm, qw_buf)

        def row0_step(v, _):
            co = v * _LANES
            val = qw_buf[0, pl.ds(co, _LANES)] + pw_buf[0, pl.ds(co, _LANES)]
            for b in range(_NB):
                small[b, 0, pl.ds(co, _LANES)] = val
            return 0

        lax.fori_loop(0, _VECS, row0_step, 0)

        def in_dma(g):
            return pltpu.async_copy(
                feats_hbm.at[pl.ds(g * _NB, _NB), pl.ds(0, 8), :],
                bufs[g % _NBUF].at[:, pl.ds(0, 8), :], sems_in[g % _NBUF])

        def out_dma(g):
            return pltpu.async_copy(
                small.at[:, pl.ds(0, 8), :],
                out_hbm.at[pl.ds(g * _NB, _NB), pl.ds(0, 8), :],
                sems_out[0])

        def compute(g):
            buf = bufs[g % _NBUF]

            def vec_step(v, _):
                co = v * _LANES
                for r in range(1, 8):
                    pwv = pw_buf[r, pl.ds(co, _LANES)]
                    for b in range(_NB):
                        small[b, r, pl.ds(co, _LANES)] = (
                            buf[b, r - 1, pl.ds(co, _LANES)] + pwv)
                return 0

            lax.fori_loop(0, _VECS, vec_step, 0)

        h_in = [None] * _NGROUPS
        h_out = [None] * _NGROUPS
        h_in[0] = in_dma(0)
        for g in range(_NGROUPS):
            if g + 1 < _NGROUPS:
                h_in[g + 1] = in_dma(g + 1)
            h_in[g].wait()
            if g >= 1:
                h_out[g - 1].wait()
            compute(g)
            h_out[g] = out_dma(g)
        h_out[_NGROUPS - 1].wait()

    @pl.when(wid == 24)
    def _tail():
        pltpu.sync_copy(pw_hbm.at[pl.ds(192, 5), :], tail_pw)

        def in_dma(g):
            return pltpu.async_copy(
                feats_hbm.at[pl.ds(g * _NB, _NB), pl.ds(184, 12), :],
                small, sems_in[0])

        def out_dma(g):
            return pltpu.async_copy(
                tail_out,
                out_hbm.at[pl.ds(g * _NB, _NB), pl.ds(192, 5), :],
                sems_out[0])

        def compute(g):
            def vec_step(v, _):
                co = v * _LANES
                for r in range(5):
                    pwv = tail_pw[r, pl.ds(co, _LANES)]
                    for b in range(_NB):
                        tail_out[b, r, pl.ds(co, _LANES)] = (
                            small[b, r + 7, pl.ds(co, _LANES)] + pwv)
                return 0

            lax.fori_loop(0, _VECS, vec_step, 0)

        h_in = [None] * _NGROUPS
        h_out = [None] * _NGROUPS
        h_in[0] = in_dma(0)
        for g in range(_NGROUPS):
            h_in[g].wait()
            if g >= 1:
                h_out[g - 1].wait()
            compute(g)
            h_out[g] = out_dma(g)
            if g + 1 < _NGROUPS:
                h_in[g + 1] = in_dma(g + 1)
        h_out[_NGROUPS - 1].wait()


@jax.jit
def kernel(feats, quality_weight, position_weight):
    mesh = plsc.VectorSubcoreMesh(core_axis_name="c", subcore_axis_name="s")
    run = pl.kernel(
        _body,
        out_type=jax.ShapeDtypeStruct((_BATCH, _P_OUT, _HIDDEN), jnp.float32),
        mesh=mesh,
        scratch_types=[
            pltpu.VMEM((_NB, 16, _HIDDEN), jnp.float32),
            pltpu.VMEM((_NB, 16, _HIDDEN), jnp.float32),
            pltpu.VMEM((_NB, 16, _HIDDEN), jnp.float32),
            pltpu.VMEM((8, _HIDDEN), jnp.float32),
            pltpu.VMEM((_NB, 12, _HIDDEN), jnp.float32),
            pltpu.VMEM((_NB, 5, _HIDDEN), jnp.float32),
            pltpu.VMEM((5, _HIDDEN), jnp.float32),
            pltpu.VMEM((1, _HIDDEN), jnp.float32),
            pltpu.SemaphoreType.DMA,
            pltpu.SemaphoreType.DMA,
            pltpu.SemaphoreType.DMA,
            pltpu.SemaphoreType.DMA,
            pltpu.SemaphoreType.DMA,
            pltpu.SemaphoreType.DMA,
        ],
        compiler_params=pltpu.CompilerParams(use_tc_tiling_on_sc=True),
    )
    return run(feats, quality_weight, position_weight)


# speedup vs baseline: 2.1503x; 2.1503x over previous
import jax
import jax.numpy as jnp
from jax import lax
from jax.experimental import pallas as pl
from jax.experimental.pallas import tpu as pltpu
from jax.experimental.pallas import tpu_sc as plsc

_BATCH = 64
_NUM_POS = 196
_HIDDEN = 768
_P_OUT = _NUM_POS + 1

_LANES = 16
_ROWS = 7
_MAIN = 28
_NB = 8
_NGROUPS = _BATCH // _NB
_VECS = _HIDDEN // _LANES


def _body(feats_hbm, qw_hbm, pw_hbm, out_hbm,
          buf0, buf1, pwin, pw5, qw_buf, row0_buf,
          si0, si1, so0, so1):
    bufs = [buf0, buf1]
    sems_in = [si0, si1]
    sems_out = [so0, so1]
    c = lax.axis_index("c")
    s = lax.axis_index("s")
    wid = s * 2 + c

    def in_dma(g, p0):
        return pltpu.async_copy(
            feats_hbm.at[pl.ds(p0, _ROWS), pl.ds(g * _NB, _NB), :],
            bufs[g % 2], sems_in[g % 2])

    def out_dma(g, p0):
        return pltpu.async_copy(
            bufs[g % 2],
            out_hbm.at[pl.ds(p0 + 1, _ROWS), pl.ds(g * _NB, _NB), :],
            sems_out[g % 2])

    def run_groups(p0, compute):
        h_in = [None] * _NGROUPS
        h_out = [None] * _NGROUPS
        h_in[0] = in_dma(0, p0)
        h_in[1] = in_dma(1, p0)
        for g in range(_NGROUPS):
            h_in[g].wait()
            compute(g)
            h_out[g] = out_dma(g, p0)
            if g >= 1 and g + 1 < _NGROUPS:
                h_out[g - 1].wait()
                h_in[g + 1] = in_dma(g + 1, p0)
        h_out[_NGROUPS - 2].wait()
        h_out[_NGROUPS - 1].wait()

    @pl.when(wid < _MAIN - 1)
    def _main():
        p0 = wid * _ROWS
        a0 = ((p0 + 1) // 8) * 8
        widx = p0 + 1 - a0
        pltpu.sync_copy(pw_hbm.at[pl.ds(a0, 16), :], pwin)

        def compute(g):
            buf = bufs[g % 2]

            def vec_step(v, _):
                co = v * _LANES
                for r in range(_ROWS):
                    pwv = pwin[widx + r, pl.ds(co, _LANES)]
                    for b in range(_NB):
                        buf[r, b, pl.ds(co, _LANES)] = (
                            buf[r, b, pl.ds(co, _LANES)] + pwv)
                return 0

            lax.fori_loop(0, _VECS, vec_step, 0)

        run_groups(p0, compute)

    @pl.when(wid == _MAIN - 1)
    def _last():
        p0 = (_MAIN - 1) * _ROWS
        pltpu.sync_copy(pw_hbm.at[pl.ds(184, 8), :], pwin.at[pl.ds(0, 8), :])
        pltpu.sync_copy(pw_hbm.at[pl.ds(192, 5), :], pw5)

        def compute(g):
            buf = bufs[g % 2]

            def vec_step(v, _):
                co = v * _LANES
                for r in range(_ROWS):
                    if r < 2:
                        pwv = pwin[6 + r, pl.ds(co, _LANES)]
                    else:
                        pwv = pw5[r - 2, pl.ds(co, _LANES)]
                    for b in range(_NB):
                        buf[r, b, pl.ds(co, _LANES)] = (
                            buf[r, b, pl.ds(co, _LANES)] + pwv)
                return 0

            lax.fori_loop(0, _VECS, vec_step, 0)

        run_groups(p0, compute)

    @pl.when(wid >= _MAIN)
    def _row0():
        b0 = (wid - _MAIN) * 16
        pltpu.sync_copy(pw_hbm.at[pl.ds(0, 8), :], pwin.at[pl.ds(0, 8), :])
        pltpu.sync_copy(qw_hbm, qw_buf)

        def vec_step(v, _):
            co = v * _LANES
            val = qw_buf[0, pl.ds(co, _LANES)] + pwin[0, pl.ds(co, _LANES)]
            for b in range(_NB):
                row0_buf[0, b, pl.ds(co, _LANES)] = val
            return 0

        lax.fori_loop(0, _VECS, vec_step, 0)
        h0 = pltpu.async_copy(
            row0_buf, out_hbm.at[pl.ds(0, 1), pl.ds(b0, _NB), :], sems_out[0])
        h1 = pltpu.async_copy(
            row0_buf, out_hbm.at[pl.ds(0, 1), pl.ds(b0 + 8, _NB), :], sems_out[1])
        h0.wait()
        h1.wait()


@jax.jit
def kernel(feats, quality_weight, position_weight):
    feats_t = jnp.transpose(feats, (1, 0, 2))
    mesh = plsc.VectorSubcoreMesh(core_axis_name="c", subcore_axis_name="s")
    run = pl.kernel(
        _body,
        out_type=jax.ShapeDtypeStruct((_P_OUT, _BATCH, _HIDDEN), jnp.float32),
        mesh=mesh,
        scratch_types=[
            pltpu.VMEM((_ROWS, _NB, _HIDDEN), jnp.float32),
            pltpu.VMEM((_ROWS, _NB, _HIDDEN), jnp.float32),
            pltpu.VMEM((16, _HIDDEN), jnp.float32),
            pltpu.VMEM((5, _HIDDEN), jnp.float32),
            pltpu.VMEM((1, _HIDDEN), jnp.float32),
            pltpu.VMEM((1, _NB, _HIDDEN), jnp.float32),
            pltpu.SemaphoreType.DMA,
            pltpu.SemaphoreType.DMA,
            pltpu.SemaphoreType.DMA,
            pltpu.SemaphoreType.DMA,
        ],
        compiler_params=pltpu.CompilerParams(use_tc_tiling_on_sc=True),
    )
    out_t = run(feats_t, quality_weight, position_weight)
    return jnp.transpose(out_t, (1, 0, 2))
